# baseline (speedup 1.0000x reference)
import functools
import jax
import jax.numpy as jnp
from jax import lax
from jax.experimental import pallas as pl
from jax.experimental.pallas import tpu as pltpu, tpu_sc as plsc

B_ROWS = 131072
F = 128
NVREG = F // 16

_info = plsc.get_sparse_core_info()
NC, NS, L = _info.num_cores, _info.num_subcores, _info.num_lanes
NW = NC * NS
RW = B_ROWS // NW
BR = 128
NBLK = RW // BR


_GDN = lax.GatherDimensionNumbers(
    offset_dims=(), collapsed_slice_dims=(0,), start_index_map=(0,)
)


def _shuffle(v, idx):
    return lax.gather(
        v,
        idx[:, None],
        dimension_numbers=_GDN,
        slice_sizes=(1,),
        mode=lax.GatherScatterMode.PROMISE_IN_BOUNDS,
    )


def _vreduce(v, lane):
    for k in (1, 2, 4, 8):
        v = v + _shuffle(v, lane ^ k)
    return v


def _sc_body(x_hbm, wb_hbm, out_hbm, wv, inb, outb, si0, si1, so0, so1):
    c = lax.axis_index("c")
    s = lax.axis_index("s")
    wid = s * NC + c
    base = wid * RW
    sins = (si0, si1)
    souts = (so0, so1)

    pltpu.sync_copy(wb_hbm, wv)
    wregs = [wv[pl.ds(16 * j, 16)] for j in range(NVREG)]
    lane = lax.iota(jnp.int32, 16)
    lane15 = lane == 15
    b_all = _vreduce(wv[pl.ds(F, 16)], lane)

    def in_copy(i, buf):
        return pltpu.make_async_copy(
            x_hbm.at[pl.ds(base + i * BR, BR)], inb.at[buf], sins[buf]
        )

    def out_copy(i, buf):
        return pltpu.make_async_copy(
            outb.at[buf], out_hbm.at[pl.ds(base + i * BR, BR)], souts[buf]
        )

    def compute(buf):
        ib = inb.at[buf]
        ob = outb.at[buf]

        def row_body(r, carry):
            prods = []
            z_last = None
            m_last = None
            v_last = None
            for j in range(NVREG):
                v = ib[r, pl.ds(16 * j, 16)]
                m = v != v
                z = jnp.where(m, 0.0, v)
                prods.append(z * wregs[j])
                if j < NVREG - 1:
                    ob[r, pl.ds(16 * j, 16)] = z
                else:
                    z_last, m_last, v_last = z, m, v
            while len(prods) > 1:
                prods = [
                    prods[k] + prods[k + 1] for k in range(0, len(prods) - 1, 2)
                ] + ([prods[-1]] if len(prods) % 2 else [])
            pred_v = _vreduce(prods[0], lane) + b_all
            out7 = jnp.where(lane15, jnp.where(m_last, pred_v, v_last), z_last)
            ob[r, pl.ds(F - 16, 16)] = out7
            return carry

        lax.fori_loop(0, BR, row_body, 0, unroll=2)

    in_copy(0, 0).start()
    in_copy(1, 1).start()

    def g_body(g, carry):
        for buf in range(2):
            i = 2 * g + buf
            in_copy(i, buf).wait()

            @pl.when(g >= 1)
            def _():
                out_copy(i - 2, buf).wait()

            compute(buf)
            out_copy(i, buf).start()

            @pl.when(g < NBLK // 2 - 1)
            def _():
                in_copy(i + 2, buf).start()

        return carry

    lax.fori_loop(0, NBLK // 2, g_body, 0, unroll=False)
    out_copy(NBLK - 2, 0).wait()
    out_copy(NBLK - 1, 1).wait()


@jax.jit
def _run(x, wb):
    mesh = plsc.VectorSubcoreMesh(core_axis_name="c", subcore_axis_name="s")
    f = functools.partial(
        pl.kernel,
        out_type=jax.ShapeDtypeStruct((B_ROWS, F), jnp.float32),
        mesh=mesh,
        scratch_types=[
            pltpu.VMEM((F + 16,), jnp.float32),
            pltpu.VMEM((2, BR, F), jnp.float32),
            pltpu.VMEM((2, BR, F), jnp.float32),
            pltpu.SemaphoreType.DMA,
            pltpu.SemaphoreType.DMA,
            pltpu.SemaphoreType.DMA,
            pltpu.SemaphoreType.DMA,
        ],
    )(_sc_body)
    return f(x, wb)


def kernel(x, W, b):
    wb = jnp.concatenate(
        [W, jnp.broadcast_to(b, (1,)), jnp.zeros((15,), jnp.float32)]
    )
    return _run(x, wb)


# speedup vs baseline: 1.6459x; 1.6459x over previous
import functools
import jax
import jax.numpy as jnp
from jax import lax
from jax.experimental import pallas as pl
from jax.experimental.pallas import tpu as pltpu, tpu_sc as plsc

B_ROWS = 131072
F = 128
NVREG = F // 16

_info = plsc.get_sparse_core_info()
NC, NS, L = _info.num_cores, _info.num_subcores, _info.num_lanes
NW = NC * NS
RW = B_ROWS // NW
BR = 128
NBLK = RW // BR


_GDN = lax.GatherDimensionNumbers(
    offset_dims=(), collapsed_slice_dims=(0,), start_index_map=(0,)
)


def _shuffle(v, idx):
    return lax.gather(
        v,
        idx[:, None],
        dimension_numbers=_GDN,
        slice_sizes=(1,),
        mode=lax.GatherScatterMode.PROMISE_IN_BOUNDS,
    )


def _vreduce(v, lane):
    for k in (1, 2, 4, 8):
        v = v + _shuffle(v, lane ^ k)
    return v


def _sc_body(x_hbm, wb_hbm, out_hbm, wv, inb, outb, si0, si1, so0, so1):
    c = lax.axis_index("c")
    s = lax.axis_index("s")
    wid = s * NC + c
    base = wid * RW
    sins = (si0, si1)
    souts = (so0, so1)

    pltpu.sync_copy(wb_hbm, wv)
    wregs = [wv[pl.ds(16 * j, 16)] for j in range(NVREG)]
    lane = lax.iota(jnp.int32, 16)
    lane15 = lane == 15
    b_all = _vreduce(wv[pl.ds(F, 16)], lane)

    def in_copy(i, buf):
        return pltpu.make_async_copy(
            x_hbm.at[pl.ds(base + i * BR, BR)], inb.at[buf], sins[buf]
        )

    def out_copy(i, buf):
        return pltpu.make_async_copy(
            outb.at[buf], out_hbm.at[pl.ds(base + i * BR, BR)], souts[buf]
        )

    def compute(buf):
        ib = inb.at[buf]
        ob = outb.at[buf]

        def row_body(r, carry):
            prods = []
            z_last = None
            m_last = None
            v_last = None
            for j in range(NVREG):
                v = ib[r, pl.ds(16 * j, 16)]
                m = v != v
                z = jnp.where(m, 0.0, v)
                prods.append(z * wregs[j])
                if j < NVREG - 1:
                    ob[r, pl.ds(16 * j, 16)] = z
                else:
                    z_last, m_last, v_last = z, m, v
            while len(prods) > 1:
                prods = [
                    prods[k] + prods[k + 1] for k in range(0, len(prods) - 1, 2)
                ] + ([prods[-1]] if len(prods) % 2 else [])
            pred_v = _vreduce(prods[0], lane) + b_all
            out7 = jnp.where(lane15, jnp.where(m_last, pred_v, v_last), z_last)
            ob[r, pl.ds(F - 16, 16)] = out7
            return carry

        lax.fori_loop(0, BR, row_body, 0, unroll=False)

    in_copy(0, 0).start()
    in_copy(1, 1).start()

    def g_body(g, carry):
        for buf in range(2):
            i = 2 * g + buf
            in_copy(i, buf).wait()

            @pl.when(g >= 1)
            def _():
                out_copy(i - 2, buf).wait()

            compute(buf)
            out_copy(i, buf).start()

            @pl.when(g < NBLK // 2 - 1)
            def _():
                in_copy(i + 2, buf).start()

        return carry

    lax.fori_loop(0, NBLK // 2, g_body, 0, unroll=False)
    out_copy(NBLK - 2, 0).wait()
    out_copy(NBLK - 1, 1).wait()


@jax.jit
def _run(x, wb):
    mesh = plsc.VectorSubcoreMesh(core_axis_name="c", subcore_axis_name="s")
    f = functools.partial(
        pl.kernel,
        out_type=jax.ShapeDtypeStruct((B_ROWS, F), jnp.float32),
        mesh=mesh,
        scratch_types=[
            pltpu.VMEM((F + 16,), jnp.float32),
            pltpu.VMEM((2, BR, F), jnp.float32),
            pltpu.VMEM((2, BR, F), jnp.float32),
            pltpu.SemaphoreType.DMA,
            pltpu.SemaphoreType.DMA,
            pltpu.SemaphoreType.DMA,
            pltpu.SemaphoreType.DMA,
        ],
    )(_sc_body)
    return f(x, wb)


def kernel(x, W, b):
    wb = jnp.concatenate(
        [W, jnp.broadcast_to(b, (1,)), jnp.zeros((15,), jnp.float32)]
    )
    return _run(x, wb)
